# baseline (speedup 1.0000x reference)
import functools

import jax
import jax.numpy as jnp
from jax import lax
from jax.experimental import pallas as pl
from jax.experimental.pallas import tpu as pltpu
from jax.experimental.pallas import tpu_sc as plsc

EMB = 32
BATCH = 16384
L = 16
TILE_C = 128
CH_COLS = 16
CHUNK_W = CH_COLS * TILE_C
N_ROWS = 1000000
N_MAIN = (N_ROWS // TILE_C) * TILE_C
N_TAIL = N_ROWS - N_MAIN
TB = EMB * N_MAIN


@functools.cache
def _build_detile():
    mesh = plsc.VectorSubcoreMesh(core_axis_name="c", subcore_axis_name="s")
    NS = 16
    n_full = N_MAIN // CHUNK_W
    per_tec = -(-n_full // NS)
    rem_w = N_MAIN - n_full * CHUNK_W
    SW = 8 * CHUNK_W

    @functools.partial(
        pl.kernel,
        mesh=mesh,
        compiler_params=pltpu.CompilerParams(
            needs_layout_passes=False, use_tc_tiling_on_sc=True),
        out_type=(jax.ShapeDtypeStruct((EMB * N_ROWS,), jnp.float32),
                  jax.ShapeDtypeStruct((EMB * N_ROWS,), jnp.float32)),
        scratch_types=[
            pltpu.VMEM((2, 8, CHUNK_W), jnp.float32),
            pltpu.VMEM((2 * SW,), jnp.float32),
            pltpu.VMEM((N_TAIL, EMB), jnp.float32),
            pltpu.VMEM((EMB * N_TAIL,), jnp.float32),
            pltpu.SemaphoreType.DMA,
            pltpu.SemaphoreType.DMA,
        ],
    )
    def k(uT_hbm, bT_hbm, utail_hbm, btail_hbm, uout_hbm, bout_hbm,
          buf, stage, tail_v, tstage, rsem, wsem):
        core = lax.axis_index("c")
        t = lax.axis_index("s")

        def drain(dst_hbm, par, words):
            pltpu.make_async_copy(
                dst_hbm.at[pl.ds(0, words)],
                stage.at[pl.ds(par * SW, words)], wsem).wait()

        def vec_copy(par, ii, words):
            def body(kk, carry):
                base = pl.multiple_of(kk * TILE_C, TILE_C)
                for q in range(TILE_C // L):
                    stage[pl.ds(par * SW + ii * CHUNK_W + base + q * L, L)] = (
                        buf[par, ii, pl.ds(base + q * L, L)])
                return carry
            lax.fori_loop(0, words // TILE_C, body, 0)

        def unit(src, dst, par, i, col0, words, do_drain):
            pltpu.async_copy(
                src.at[pl.ds(8 * i, 8), pl.ds(col0, words)],
                buf.at[par, :, pl.ds(0, words)], rsem).wait()
            if do_drain is not None:
                do_drain()
            for ii in range(8):
                vec_copy(par, ii, words)
            for ii in range(8):
                pltpu.async_copy(
                    stage.at[pl.ds(par * SW + ii * CHUNK_W, words)],
                    dst.at[pl.ds((8 * i + ii) * N_MAIN + col0, words)],
                    wsem)

        def detile_one(src, dst, tail_src):
            def chunk_body(g, carry):
                chunk = g * NS + t

                @pl.when(chunk < n_full)
                def _():
                    for i in range(4):
                        def cond_drain(p=i % 2, always=(i >= 2)):
                            if always:
                                drain(dst, p, SW)
                            else:
                                @pl.when(g >= 1)
                                def _():
                                    drain(dst, p, SW)
                        unit(src, dst, i % 2, i, chunk * CHUNK_W, CHUNK_W,
                             cond_drain)
                return carry

            lax.fori_loop(0, per_tec, chunk_body, 0)
            for j in range(2):
                drain(dst, j, SW)

            @pl.when(t == NS - 1)
            def _():
                col0 = n_full * CHUNK_W
                for i in range(4):
                    unit(src, dst, i % 2, i, col0, rem_w,
                         (lambda p=i % 2: drain(dst, p, 8 * rem_w))
                         if i >= 2 else None)
                for j in range(2):
                    drain(dst, j, 8 * rem_w)

            @pl.when(t == NS - 2)
            def _():
                pltpu.async_copy(tail_src, tail_v, rsem).wait()
                lanes = lax.iota(jnp.int32, L)

                def tail_row(r, carry):
                    for h in range(EMB // L):
                        vals = tail_v[r, pl.ds(h * L, L)]
                        offs = (lanes + h * L) * N_TAIL + r
                        plsc.store_scatter(tstage, [offs], vals)
                    return carry

                lax.fori_loop(0, N_TAIL, tail_row, 0)
                pltpu.async_copy(tstage, dst.at[pl.ds(TB, EMB * N_TAIL)],
                                 wsem)
                pltpu.make_async_copy(dst.at[pl.ds(0, EMB * N_TAIL)],
                                      tstage, wsem).wait()

        @pl.when(core == 0)
        def _():
            detile_one(uT_hbm, uout_hbm, utail_hbm)

        @pl.when(core == 1)
        def _():
            detile_one(bT_hbm, bout_hbm, btail_hbm)

    return k


@functools.cache
def _build_gather():
    mesh = plsc.VectorSubcoreMesh(core_axis_name="c", subcore_axis_name="s")
    NC, NS = 2, 16
    NW = NC * NS
    BW = BATCH // NW

    @functools.partial(
        pl.kernel,
        mesh=mesh,
        compiler_params=pltpu.CompilerParams(needs_layout_passes=False),
        out_type=jax.ShapeDtypeStruct((BATCH,), jnp.float32),
        scratch_types=[
            pltpu.VMEM((BW,), jnp.int32),
            pltpu.VMEM((BW,), jnp.int32),
            pltpu.VMEM((EMB * BW,), jnp.int32),
            pltpu.VMEM((EMB * BW,), jnp.int32),
            pltpu.VMEM((EMB * BW,), jnp.float32),
            pltpu.VMEM((EMB * BW,), jnp.float32),
            pltpu.VMEM((BW,), jnp.float32),
            pltpu.VMEM((BW,), jnp.float32),
            pltpu.VMEM((BW,), jnp.float32),
            pltpu.SemaphoreType.DMA,
            pltpu.SemaphoreType.DMA,
        ],
    )
    def k(uidx_hbm, bidx_hbm, uflat_hbm, ubias_hbm, bflat_hbm, bbias_hbm,
          out_hbm, uidx_v, bidx_v, uoffs_v, boffs_v, uval_v, bval_v,
          ubias_v, bbias_v, res_v, gsem, bsem):
        wid = lax.axis_index("s") * NC + lax.axis_index("c")
        base = wid * BW

        pltpu.sync_copy(uidx_hbm.at[pl.ds(base, BW)], uidx_v)
        pltpu.sync_copy(bidx_hbm.at[pl.ds(base, BW)], bidx_v)

        cb0 = pltpu.async_copy(ubias_hbm.at[uidx_v], ubias_v, bsem)
        cb1 = pltpu.async_copy(bbias_hbm.at[bidx_v], bbias_v, bsem)

        def build(idx_v, offs_v):
            def body(g, carry):
                idxg = idx_v[pl.ds(g * L, L)]
                in_main = idxg < N_MAIN
                for d in range(EMB):
                    delta = jnp.where(in_main, d * N_MAIN,
                                      TB + d * N_TAIL - N_MAIN)
                    offs_v[pl.ds(d * BW + g * L, L)] = idxg + delta
                return carry
            lax.fori_loop(0, BW // L, body, 0)

        build(uidx_v, uoffs_v)
        cu = pltpu.async_copy(uflat_hbm.at[uoffs_v], uval_v, gsem)
        build(bidx_v, boffs_v)
        cb = pltpu.async_copy(bflat_hbm.at[boffs_v], bval_v, gsem)
        cu.wait()
        cb.wait()
        cb0.wait()
        cb1.wait()

        def group(g, carry):
            s = pl.ds(g * L, L)
            acc = ubias_v[s] + bbias_v[s]
            for d in range(EMB):
                sd = pl.ds(d * BW + g * L, L)
                acc = acc + uval_v[sd] * bval_v[sd]
            res_v[s] = 1.0 / (1.0 + jnp.exp(-acc))
            return carry

        lax.fori_loop(0, BW // L, group, 0)
        pltpu.sync_copy(res_v, out_hbm.at[pl.ds(base, BW)])

    return k


def kernel(inputs, user_emb, user_bias, book_emb, book_bias):
    uflat, bflat = _build_detile()(
        user_emb.T, book_emb.T,
        user_emb[N_MAIN:, :], book_emb[N_MAIN:, :])
    out = _build_gather()(
        inputs[:, 0], inputs[:, 1], uflat, user_bias.reshape(-1),
        bflat, book_bias.reshape(-1))
    return out.reshape(BATCH, 1)


# speedup vs baseline: 1.3740x; 1.3740x over previous
import functools

import jax
import jax.numpy as jnp
from jax import lax
from jax.experimental import pallas as pl
from jax.experimental.pallas import tpu as pltpu
from jax.experimental.pallas import tpu_sc as plsc

EMB = 32
BATCH = 16384
L = 16
TILE_C = 128
CH_COLS = 16
CHUNK_W = CH_COLS * TILE_C
N_ROWS = 1000000
N_MAIN = (N_ROWS // TILE_C) * TILE_C
N_TAIL = N_ROWS - N_MAIN
TB = EMB * N_MAIN


@functools.cache
def _build_detile():
    mesh = plsc.VectorSubcoreMesh(core_axis_name="c", subcore_axis_name="s")
    NS = 16
    n_full = N_MAIN // CHUNK_W
    per_tec = -(-n_full // NS)
    rem_w = N_MAIN - n_full * CHUNK_W
    SW = 8 * CHUNK_W

    @functools.partial(
        pl.kernel,
        mesh=mesh,
        compiler_params=pltpu.CompilerParams(
            needs_layout_passes=False, use_tc_tiling_on_sc=True),
        out_type=(jax.ShapeDtypeStruct((EMB * N_ROWS,), jnp.float32),
                  jax.ShapeDtypeStruct((EMB * N_ROWS,), jnp.float32)),
        scratch_types=[
            pltpu.VMEM((2, 8, CHUNK_W), jnp.float32),
            pltpu.VMEM((2 * SW,), jnp.float32),
            pltpu.VMEM((N_TAIL, EMB), jnp.float32),
            pltpu.VMEM((EMB * N_TAIL,), jnp.float32),
            pltpu.SemaphoreType.DMA,
            pltpu.SemaphoreType.DMA,
        ],
    )
    def k(uT_hbm, bT_hbm, utail_hbm, btail_hbm, uout_hbm, bout_hbm,
          buf, stage, tail_v, tstage, rsem, wsem):
        core = lax.axis_index("c")
        t = lax.axis_index("s")

        def drain(dst_hbm, par, words):
            pltpu.make_async_copy(
                dst_hbm.at[pl.ds(0, words)],
                stage.at[pl.ds(par * SW, words)], wsem).wait()

        def vec_copy(par, ii, words):
            def body(kk, carry):
                base = pl.multiple_of(kk * TILE_C, TILE_C)
                for q in range(TILE_C // L):
                    stage[pl.ds(par * SW + ii * CHUNK_W + base + q * L, L)] = (
                        buf[par, ii, pl.ds(base + q * L, L)])
                return carry
            lax.fori_loop(0, words // TILE_C, body, 0)

        def read_unit(src, par, i, col0):
            pltpu.async_copy(
                src.at[pl.ds(8 * i, 8), pl.ds(col0, CHUNK_W)],
                buf.at[par], rsem)

        def rwait(src, par):
            pltpu.make_async_copy(
                src.at[pl.ds(0, 8), pl.ds(0, CHUNK_W)],
                buf.at[par], rsem).wait()

        def unit(src, dst, par, i, col0, words, do_drain):
            pltpu.async_copy(
                src.at[pl.ds(8 * i, 8), pl.ds(col0, words)],
                buf.at[par, :, pl.ds(0, words)], rsem).wait()
            if do_drain is not None:
                do_drain()
            for ii in range(8):
                vec_copy(par, ii, words)
            for ii in range(8):
                pltpu.async_copy(
                    stage.at[pl.ds(par * SW + ii * CHUNK_W, words)],
                    dst.at[pl.ds((8 * i + ii) * N_MAIN + col0, words)],
                    wsem)

        def detile_one(src, dst, tail_src):
            @pl.when(t < n_full)
            def _():
                read_unit(src, 0, 0, t * CHUNK_W)
                read_unit(src, 1, 1, t * CHUNK_W)

            def chunk_body(g, carry):
                chunk = g * NS + t
                nchunk = (g + 1) * NS + t

                @pl.when(chunk < n_full)
                def _():
                    col0 = chunk * CHUNK_W
                    ncol0 = nchunk * CHUNK_W
                    for i in range(4):
                        par = i % 2
                        rwait(src, par)
                        if i >= 2:
                            drain(dst, par, SW)
                        else:
                            @pl.when(g >= 1)
                            def _():
                                drain(dst, par, SW)
                        for ii in range(8):
                            vec_copy(par, ii, CHUNK_W)
                        for ii in range(8):
                            pltpu.async_copy(
                                stage.at[pl.ds(par * SW + ii * CHUNK_W,
                                               CHUNK_W)],
                                dst.at[pl.ds((8 * i + ii) * N_MAIN + col0,
                                             CHUNK_W)],
                                wsem)
                        if i < 2:
                            read_unit(src, par, i + 2, col0)
                        else:
                            @pl.when(nchunk < n_full)
                            def _():
                                read_unit(src, par, i - 2, ncol0)
                return carry

            lax.fori_loop(0, per_tec, chunk_body, 0)
            for j in range(2):
                drain(dst, j, SW)

            @pl.when(t == NS - 1)
            def _():
                col0 = n_full * CHUNK_W
                for i in range(4):
                    unit(src, dst, i % 2, i, col0, rem_w,
                         (lambda p=i % 2: drain(dst, p, 8 * rem_w))
                         if i >= 2 else None)
                for j in range(2):
                    drain(dst, j, 8 * rem_w)

            @pl.when(t == NS - 2)
            def _():
                pltpu.async_copy(tail_src, tail_v, rsem).wait()
                lanes = lax.iota(jnp.int32, L)

                def tail_row(r, carry):
                    for h in range(EMB // L):
                        vals = tail_v[r, pl.ds(h * L, L)]
                        offs = (lanes + h * L) * N_TAIL + r
                        plsc.store_scatter(tstage, [offs], vals)
                    return carry

                lax.fori_loop(0, N_TAIL, tail_row, 0)
                pltpu.async_copy(tstage, dst.at[pl.ds(TB, EMB * N_TAIL)],
                                 wsem)
                pltpu.make_async_copy(dst.at[pl.ds(0, EMB * N_TAIL)],
                                      tstage, wsem).wait()

        @pl.when(core == 0)
        def _():
            detile_one(uT_hbm, uout_hbm, utail_hbm)

        @pl.when(core == 1)
        def _():
            detile_one(bT_hbm, bout_hbm, btail_hbm)

    return k


@functools.cache
def _build_gather():
    mesh = plsc.VectorSubcoreMesh(core_axis_name="c", subcore_axis_name="s")
    NC, NS = 2, 16
    NW = NC * NS
    BW = BATCH // NW

    @functools.partial(
        pl.kernel,
        mesh=mesh,
        compiler_params=pltpu.CompilerParams(needs_layout_passes=False),
        out_type=jax.ShapeDtypeStruct((BATCH,), jnp.float32),
        scratch_types=[
            pltpu.VMEM((BW,), jnp.int32),
            pltpu.VMEM((BW,), jnp.int32),
            pltpu.VMEM((EMB * BW,), jnp.int32),
            pltpu.VMEM((EMB * BW,), jnp.int32),
            pltpu.VMEM((EMB * BW,), jnp.float32),
            pltpu.VMEM((EMB * BW,), jnp.float32),
            pltpu.VMEM((BW,), jnp.float32),
            pltpu.VMEM((BW,), jnp.float32),
            pltpu.VMEM((BW,), jnp.float32),
            pltpu.SemaphoreType.DMA,
            pltpu.SemaphoreType.DMA,
        ],
    )
    def k(uidx_hbm, bidx_hbm, uflat_hbm, ubias_hbm, bflat_hbm, bbias_hbm,
          out_hbm, uidx_v, bidx_v, uoffs_v, boffs_v, uval_v, bval_v,
          ubias_v, bbias_v, res_v, gsem, bsem):
        wid = lax.axis_index("s") * NC + lax.axis_index("c")
        base = wid * BW

        pltpu.sync_copy(uidx_hbm.at[pl.ds(base, BW)], uidx_v)
        pltpu.sync_copy(bidx_hbm.at[pl.ds(base, BW)], bidx_v)

        cb0 = pltpu.async_copy(ubias_hbm.at[uidx_v], ubias_v, bsem)
        cb1 = pltpu.async_copy(bbias_hbm.at[bidx_v], bbias_v, bsem)

        def build(idx_v, offs_v):
            def body(g, carry):
                idxg = idx_v[pl.ds(g * L, L)]
                in_main = idxg < N_MAIN
                for d in range(EMB):
                    delta = jnp.where(in_main, d * N_MAIN,
                                      TB + d * N_TAIL - N_MAIN)
                    offs_v[pl.ds(d * BW + g * L, L)] = idxg + delta
                return carry
            lax.fori_loop(0, BW // L, body, 0)

        build(uidx_v, uoffs_v)
        cu = pltpu.async_copy(uflat_hbm.at[uoffs_v], uval_v, gsem)
        build(bidx_v, boffs_v)
        cb = pltpu.async_copy(bflat_hbm.at[boffs_v], bval_v, gsem)
        cu.wait()
        cb.wait()
        cb0.wait()
        cb1.wait()

        def group(g, carry):
            s = pl.ds(g * L, L)
            acc = ubias_v[s] + bbias_v[s]
            for d in range(EMB):
                sd = pl.ds(d * BW + g * L, L)
                acc = acc + uval_v[sd] * bval_v[sd]
            res_v[s] = 1.0 / (1.0 + jnp.exp(-acc))
            return carry

        lax.fori_loop(0, BW // L, group, 0)
        pltpu.sync_copy(res_v, out_hbm.at[pl.ds(base, BW)])

    return k


def kernel(inputs, user_emb, user_bias, book_emb, book_bias):
    uflat, bflat = _build_detile()(
        user_emb.T, book_emb.T,
        user_emb[N_MAIN:, :], book_emb[N_MAIN:, :])
    out = _build_gather()(
        inputs[:, 0], inputs[:, 1], uflat, user_bias.reshape(-1),
        bflat, book_bias.reshape(-1))
    return out.reshape(BATCH, 1)


# speedup vs baseline: 2.9464x; 2.1444x over previous
import functools

import jax
import jax.numpy as jnp
from jax import lax
from jax.experimental import pallas as pl
from jax.experimental.pallas import tpu as pltpu
from jax.experimental.pallas import tpu_sc as plsc

EMB = 32
BATCH = 16384
L = 16
TILE_C = 128
CH_COLS = 16
CHUNK_W = CH_COLS * TILE_C
N_ROWS = 1000000
N_MAIN = (N_ROWS // TILE_C) * TILE_C
N_TAIL = N_ROWS - N_MAIN
TB = EMB * N_MAIN


@functools.cache
def _build_detile():
    mesh = plsc.VectorSubcoreMesh(core_axis_name="c", subcore_axis_name="s")
    NS = 16
    n_full = N_MAIN // CHUNK_W
    per_tec = -(-n_full // NS)
    rem_w = N_MAIN - n_full * CHUNK_W
    SW = 8 * CHUNK_W

    @functools.partial(
        pl.kernel,
        mesh=mesh,
        compiler_params=pltpu.CompilerParams(
            needs_layout_passes=False, use_tc_tiling_on_sc=True),
        out_type=(jax.ShapeDtypeStruct((EMB * N_ROWS,), jnp.float32),
                  jax.ShapeDtypeStruct((EMB * N_ROWS,), jnp.float32)),
        scratch_types=[
            pltpu.VMEM((2, 8, CHUNK_W), jnp.float32),
            pltpu.VMEM((2 * SW,), jnp.float32),
            pltpu.VMEM((N_TAIL, EMB), jnp.float32),
            pltpu.VMEM((EMB * N_TAIL,), jnp.float32),
            pltpu.SemaphoreType.DMA,
            pltpu.SemaphoreType.DMA,
        ],
    )
    def k(uT_hbm, bT_hbm, utail_hbm, btail_hbm, uout_hbm, bout_hbm,
          buf, stage, tail_v, tstage, rsem, wsem):
        core = lax.axis_index("c")
        t = lax.axis_index("s")

        def drain(dst_hbm, par, words):
            pltpu.make_async_copy(
                dst_hbm.at[pl.ds(0, words)],
                stage.at[pl.ds(par * SW, words)], wsem).wait()

        def vec_copy(par, ii, words):
            @functools.partial(plsc.parallel_loop, 0, words // TILE_C,
                               unroll=2)
            def _(kk):
                base = pl.multiple_of(kk * TILE_C, TILE_C)
                for q in range(TILE_C // L):
                    stage[pl.ds(par * SW + ii * CHUNK_W + base + q * L, L)] = (
                        buf[par, ii, pl.ds(base + q * L, L)])

        def read_unit(src, par, i, col0):
            pltpu.async_copy(
                src.at[pl.ds(8 * i, 8), pl.ds(col0, CHUNK_W)],
                buf.at[par], rsem)

        def rwait(src, par):
            pltpu.make_async_copy(
                src.at[pl.ds(0, 8), pl.ds(0, CHUNK_W)],
                buf.at[par], rsem).wait()

        def unit(src, dst, par, i, col0, words, do_drain):
            pltpu.async_copy(
                src.at[pl.ds(8 * i, 8), pl.ds(col0, words)],
                buf.at[par, :, pl.ds(0, words)], rsem).wait()
            if do_drain is not None:
                do_drain()
            for ii in range(8):
                vec_copy(par, ii, words)
            for ii in range(8):
                pltpu.async_copy(
                    stage.at[pl.ds(par * SW + ii * CHUNK_W, words)],
                    dst.at[pl.ds((8 * i + ii) * N_MAIN + col0, words)],
                    wsem)

        def detile_one(src, dst, tail_src):
            @pl.when(t < n_full)
            def _():
                read_unit(src, 0, 0, t * CHUNK_W)
                read_unit(src, 1, 1, t * CHUNK_W)

            def chunk_body(g, carry):
                chunk = g * NS + t
                nchunk = (g + 1) * NS + t

                @pl.when(chunk < n_full)
                def _():
                    col0 = chunk * CHUNK_W
                    ncol0 = nchunk * CHUNK_W
                    for i in range(4):
                        par = i % 2
                        rwait(src, par)
                        if i >= 2:
                            drain(dst, par, SW)
                        else:
                            @pl.when(g >= 1)
                            def _():
                                drain(dst, par, SW)
                        for ii in range(8):
                            vec_copy(par, ii, CHUNK_W)
                        for ii in range(8):
                            pltpu.async_copy(
                                stage.at[pl.ds(par * SW + ii * CHUNK_W,
                                               CHUNK_W)],
                                dst.at[pl.ds((8 * i + ii) * N_MAIN + col0,
                                             CHUNK_W)],
                                wsem)
                        if i < 2:
                            read_unit(src, par, i + 2, col0)
                        else:
                            @pl.when(nchunk < n_full)
                            def _():
                                read_unit(src, par, i - 2, ncol0)
                return carry

            lax.fori_loop(0, per_tec, chunk_body, 0)
            for j in range(2):
                drain(dst, j, SW)

            @pl.when(t == NS - 1)
            def _():
                col0 = n_full * CHUNK_W
                for i in range(4):
                    unit(src, dst, i % 2, i, col0, rem_w,
                         (lambda p=i % 2: drain(dst, p, 8 * rem_w))
                         if i >= 2 else None)
                for j in range(2):
                    drain(dst, j, 8 * rem_w)

            @pl.when(t == NS - 2)
            def _():
                pltpu.async_copy(tail_src, tail_v, rsem).wait()
                lanes = lax.iota(jnp.int32, L)

                def tail_row(r, carry):
                    for h in range(EMB // L):
                        vals = tail_v[r, pl.ds(h * L, L)]
                        offs = (lanes + h * L) * N_TAIL + r
                        plsc.store_scatter(tstage, [offs], vals)
                    return carry

                lax.fori_loop(0, N_TAIL, tail_row, 0)
                pltpu.async_copy(tstage, dst.at[pl.ds(TB, EMB * N_TAIL)],
                                 wsem)
                pltpu.make_async_copy(dst.at[pl.ds(0, EMB * N_TAIL)],
                                      tstage, wsem).wait()

        @pl.when(core == 0)
        def _():
            detile_one(uT_hbm, uout_hbm, utail_hbm)

        @pl.when(core == 1)
        def _():
            detile_one(bT_hbm, bout_hbm, btail_hbm)

    return k


@functools.cache
def _build_gather():
    mesh = plsc.VectorSubcoreMesh(core_axis_name="c", subcore_axis_name="s")
    NC, NS = 2, 16
    NW = NC * NS
    BW = BATCH // NW

    @functools.partial(
        pl.kernel,
        mesh=mesh,
        compiler_params=pltpu.CompilerParams(needs_layout_passes=False),
        out_type=jax.ShapeDtypeStruct((BATCH,), jnp.float32),
        scratch_types=[
            pltpu.VMEM((BW,), jnp.int32),
            pltpu.VMEM((BW,), jnp.int32),
            pltpu.VMEM((EMB * BW,), jnp.int32),
            pltpu.VMEM((EMB * BW,), jnp.int32),
            pltpu.VMEM((EMB * BW,), jnp.float32),
            pltpu.VMEM((EMB * BW,), jnp.float32),
            pltpu.VMEM((BW,), jnp.float32),
            pltpu.VMEM((BW,), jnp.float32),
            pltpu.VMEM((BW,), jnp.float32),
            pltpu.SemaphoreType.DMA,
            pltpu.SemaphoreType.DMA,
        ],
    )
    def k(uidx_hbm, bidx_hbm, uflat_hbm, ubias_hbm, bflat_hbm, bbias_hbm,
          out_hbm, uidx_v, bidx_v, uoffs_v, boffs_v, uval_v, bval_v,
          ubias_v, bbias_v, res_v, gsem, bsem):
        wid = lax.axis_index("s") * NC + lax.axis_index("c")
        base = wid * BW

        pltpu.sync_copy(uidx_hbm.at[pl.ds(base, BW)], uidx_v)
        pltpu.sync_copy(bidx_hbm.at[pl.ds(base, BW)], bidx_v)

        cb0 = pltpu.async_copy(ubias_hbm.at[uidx_v], ubias_v, bsem)
        cb1 = pltpu.async_copy(bbias_hbm.at[bidx_v], bbias_v, bsem)

        def build(idx_v, offs_v):
            def body(g, carry):
                idxg = idx_v[pl.ds(g * L, L)]
                in_main = idxg < N_MAIN
                for d in range(EMB):
                    delta = jnp.where(in_main, d * N_MAIN,
                                      TB + d * N_TAIL - N_MAIN)
                    offs_v[pl.ds(d * BW + g * L, L)] = idxg + delta
                return carry
            lax.fori_loop(0, BW // L, body, 0)

        build(uidx_v, uoffs_v)
        cu = pltpu.async_copy(uflat_hbm.at[uoffs_v], uval_v, gsem)
        build(bidx_v, boffs_v)
        cb = pltpu.async_copy(bflat_hbm.at[boffs_v], bval_v, gsem)
        cu.wait()
        cb.wait()
        cb0.wait()
        cb1.wait()

        def group(g, carry):
            s = pl.ds(g * L, L)
            acc = ubias_v[s] + bbias_v[s]
            for d in range(EMB):
                sd = pl.ds(d * BW + g * L, L)
                acc = acc + uval_v[sd] * bval_v[sd]
            res_v[s] = 1.0 / (1.0 + jnp.exp(-acc))
            return carry

        lax.fori_loop(0, BW // L, group, 0)
        pltpu.sync_copy(res_v, out_hbm.at[pl.ds(base, BW)])

    return k


def kernel(inputs, user_emb, user_bias, book_emb, book_bias):
    uflat, bflat = _build_detile()(
        user_emb.T, book_emb.T,
        user_emb[N_MAIN:, :], book_emb[N_MAIN:, :])
    out = _build_gather()(
        inputs[:, 0], inputs[:, 1], uflat, user_bias.reshape(-1),
        bflat, book_bias.reshape(-1))
    return out.reshape(BATCH, 1)


# speedup vs baseline: 2.9807x; 1.0116x over previous
import functools

import jax
import jax.numpy as jnp
from jax import lax
from jax.experimental import pallas as pl
from jax.experimental.pallas import tpu as pltpu
from jax.experimental.pallas import tpu_sc as plsc

EMB = 32
BATCH = 16384
L = 16
TILE_C = 128
CH_COLS = 16
CHUNK_W = CH_COLS * TILE_C
N_ROWS = 1000000
N_MAIN = (N_ROWS // TILE_C) * TILE_C
N_TAIL = N_ROWS - N_MAIN
TB = EMB * N_MAIN


@functools.cache
def _build_detile():
    mesh = plsc.VectorSubcoreMesh(core_axis_name="c", subcore_axis_name="s")
    NS = 16
    n_full = N_MAIN // CHUNK_W
    per_tec = -(-n_full // NS)
    rem_w = N_MAIN - n_full * CHUNK_W
    SW = 8 * CHUNK_W

    @functools.partial(
        pl.kernel,
        mesh=mesh,
        compiler_params=pltpu.CompilerParams(
            needs_layout_passes=False, use_tc_tiling_on_sc=True),
        out_type=(jax.ShapeDtypeStruct((EMB * N_ROWS,), jnp.float32),
                  jax.ShapeDtypeStruct((EMB * N_ROWS,), jnp.float32)),
        scratch_types=[
            pltpu.VMEM((2, 8, CHUNK_W), jnp.float32),
            pltpu.VMEM((2 * SW,), jnp.float32),
            pltpu.VMEM((N_TAIL, EMB), jnp.float32),
            pltpu.VMEM((EMB * N_TAIL,), jnp.float32),
            pltpu.SemaphoreType.DMA,
            pltpu.SemaphoreType.DMA,
        ],
    )
    def k(uT_hbm, bT_hbm, utail_hbm, btail_hbm, uout_hbm, bout_hbm,
          buf, stage, tail_v, tstage, rsem, wsem):
        core = lax.axis_index("c")
        t = lax.axis_index("s")

        def drain(dst_hbm, par, words):
            pltpu.make_async_copy(
                dst_hbm.at[pl.ds(0, words)],
                stage.at[pl.ds(par * SW, words)], wsem).wait()

        def vec_copy(par, ii, words):
            @functools.partial(plsc.parallel_loop, 0, words // TILE_C,
                               unroll=2)
            def _(kk):
                base = pl.multiple_of(kk * TILE_C, TILE_C)
                for q in range(TILE_C // L):
                    stage[pl.ds(par * SW + ii * CHUNK_W + base + q * L, L)] = (
                        buf[par, ii, pl.ds(base + q * L, L)])

        def read_unit(src, par, i, col0):
            pltpu.async_copy(
                src.at[pl.ds(8 * i, 8), pl.ds(col0, CHUNK_W)],
                buf.at[par], rsem)

        def rwait(src, par):
            pltpu.make_async_copy(
                src.at[pl.ds(0, 8), pl.ds(0, CHUNK_W)],
                buf.at[par], rsem).wait()

        def unit(src, dst, par, i, col0, words, do_drain):
            pltpu.async_copy(
                src.at[pl.ds(8 * i, 8), pl.ds(col0, words)],
                buf.at[par, :, pl.ds(0, words)], rsem).wait()
            if do_drain is not None:
                do_drain()
            for ii in range(8):
                vec_copy(par, ii, words)
            for ii in range(8):
                pltpu.async_copy(
                    stage.at[pl.ds(par * SW + ii * CHUNK_W, words)],
                    dst.at[pl.ds((8 * i + ii) * N_MAIN + col0, words)],
                    wsem)

        def detile_one(src, dst, tail_src):
            @pl.when(t < n_full)
            def _():
                read_unit(src, 0, 0, t * CHUNK_W)
                read_unit(src, 1, 1, t * CHUNK_W)

            def chunk_body(g, carry):
                chunk = g * NS + t
                nchunk = (g + 1) * NS + t

                @pl.when(chunk < n_full)
                def _():
                    col0 = chunk * CHUNK_W
                    ncol0 = nchunk * CHUNK_W
                    for i in range(4):
                        par = i % 2
                        rwait(src, par)
                        if i >= 2:
                            drain(dst, par, SW)
                        else:
                            @pl.when(g >= 1)
                            def _():
                                drain(dst, par, SW)
                        for ii in range(8):
                            vec_copy(par, ii, CHUNK_W)
                        for ii in range(8):
                            pltpu.async_copy(
                                stage.at[pl.ds(par * SW + ii * CHUNK_W,
                                               CHUNK_W)],
                                dst.at[pl.ds((8 * i + ii) * N_MAIN + col0,
                                             CHUNK_W)],
                                wsem)
                        if i < 2:
                            read_unit(src, par, i + 2, col0)
                        else:
                            @pl.when(nchunk < n_full)
                            def _():
                                read_unit(src, par, i - 2, ncol0)
                return carry

            lax.fori_loop(0, per_tec, chunk_body, 0)
            for j in range(2):
                drain(dst, j, SW)

            @pl.when(t == NS - 1)
            def _():
                col0 = n_full * CHUNK_W
                for i in range(4):
                    unit(src, dst, i % 2, i, col0, rem_w,
                         (lambda p=i % 2: drain(dst, p, 8 * rem_w))
                         if i >= 2 else None)
                for j in range(2):
                    drain(dst, j, 8 * rem_w)

            @pl.when(t == NS - 2)
            def _():
                pltpu.async_copy(tail_src, tail_v, rsem).wait()
                lanes = lax.iota(jnp.int32, L)

                def tail_row(r, carry):
                    for h in range(EMB // L):
                        vals = tail_v[r, pl.ds(h * L, L)]
                        offs = (lanes + h * L) * N_TAIL + r
                        plsc.store_scatter(tstage, [offs], vals)
                    return carry

                lax.fori_loop(0, N_TAIL, tail_row, 0)
                pltpu.async_copy(tstage, dst.at[pl.ds(TB, EMB * N_TAIL)],
                                 wsem)
                pltpu.make_async_copy(dst.at[pl.ds(0, EMB * N_TAIL)],
                                      tstage, wsem).wait()

        @pl.when(core == 0)
        def _():
            detile_one(uT_hbm, uout_hbm, utail_hbm)

        @pl.when(core == 1)
        def _():
            detile_one(bT_hbm, bout_hbm, btail_hbm)

    return k


@functools.cache
def _build_gather():
    mesh = plsc.VectorSubcoreMesh(core_axis_name="c", subcore_axis_name="s")
    NC, NS = 2, 16
    NW = NC * NS
    BW = BATCH // NW

    @functools.partial(
        pl.kernel,
        mesh=mesh,
        compiler_params=pltpu.CompilerParams(needs_layout_passes=False),
        out_type=jax.ShapeDtypeStruct((BATCH,), jnp.float32),
        scratch_types=[
            pltpu.VMEM((BW,), jnp.int32),
            pltpu.VMEM((BW,), jnp.int32),
            pltpu.VMEM((EMB * BW,), jnp.int32),
            pltpu.VMEM((EMB * BW,), jnp.int32),
            pltpu.VMEM((EMB * BW,), jnp.float32),
            pltpu.VMEM((EMB * BW,), jnp.float32),
            pltpu.VMEM((BW,), jnp.float32),
            pltpu.VMEM((BW,), jnp.float32),
            pltpu.VMEM((BW,), jnp.float32),
            pltpu.SemaphoreType.DMA,
            pltpu.SemaphoreType.DMA,
        ],
    )
    def k(uidx_hbm, bidx_hbm, uflat_hbm, ubias_hbm, bflat_hbm, bbias_hbm,
          out_hbm, uidx_v, bidx_v, uoffs_v, boffs_v, uval_v, bval_v,
          ubias_v, bbias_v, res_v, gsem, bsem):
        wid = lax.axis_index("s") * NC + lax.axis_index("c")
        base = wid * BW

        pltpu.sync_copy(uidx_hbm.at[pl.ds(base, BW)], uidx_v)
        pltpu.sync_copy(bidx_hbm.at[pl.ds(base, BW)], bidx_v)

        cb0 = pltpu.async_copy(ubias_hbm.at[uidx_v], ubias_v, bsem)
        cb1 = pltpu.async_copy(bbias_hbm.at[bidx_v], bbias_v, bsem)

        def build(idx_v, offs_v):
            @functools.partial(plsc.parallel_loop, 0, BW // L, unroll=2)
            def _(g):
                idxg = idx_v[pl.ds(g * L, L)]
                in_main = idxg < N_MAIN
                for d in range(EMB):
                    delta = jnp.where(in_main, d * N_MAIN,
                                      TB + d * N_TAIL - N_MAIN)
                    offs_v[pl.ds(d * BW + g * L, L)] = idxg + delta

        build(uidx_v, uoffs_v)
        cu = pltpu.async_copy(uflat_hbm.at[uoffs_v], uval_v, gsem)
        build(bidx_v, boffs_v)
        cb = pltpu.async_copy(bflat_hbm.at[boffs_v], bval_v, gsem)
        cu.wait()
        cb.wait()
        cb0.wait()
        cb1.wait()

        @functools.partial(plsc.parallel_loop, 0, BW // L, unroll=2)
        def _(g):
            s = pl.ds(g * L, L)
            acc = ubias_v[s] + bbias_v[s]
            for d in range(EMB):
                sd = pl.ds(d * BW + g * L, L)
                acc = acc + uval_v[sd] * bval_v[sd]
            res_v[s] = 1.0 / (1.0 + jnp.exp(-acc))

        pltpu.sync_copy(res_v, out_hbm.at[pl.ds(base, BW)])

    return k


def kernel(inputs, user_emb, user_bias, book_emb, book_bias):
    uflat, bflat = _build_detile()(
        user_emb.T, book_emb.T,
        user_emb[N_MAIN:, :], book_emb[N_MAIN:, :])
    out = _build_gather()(
        inputs[:, 0], inputs[:, 1], uflat, user_bias.reshape(-1),
        bflat, book_bias.reshape(-1))
    return out.reshape(BATCH, 1)
